# baseline (speedup 1.0000x reference)
import functools

import jax
import jax.numpy as jnp
from jax import lax
from jax.experimental import pallas as pl
from jax.experimental.pallas import tpu as pltpu
from jax.experimental.pallas import tpu_sc as plsc

N_SHIFT = 128
CHUNK = 4


def kernel(x):
    B, R, T = x.shape
    rows = B * R
    n_workers = 32
    rows_per_w = rows // n_workers
    n_chunks = rows_per_w // CHUNK
    body = T - N_SHIFT

    mesh = plsc.VectorSubcoreMesh(core_axis_name="c", subcore_axis_name="s")

    @functools.partial(
        pl.kernel,
        mesh=mesh,
        out_type=jax.ShapeDtypeStruct((rows, T), jnp.float32),
        scratch_types=[
            pltpu.VMEM((CHUNK, T), jnp.float32),
            pltpu.VMEM((CHUNK, T), jnp.float32),
            pltpu.SemaphoreType.DMA,
            pltpu.SemaphoreType.DMA,
            pltpu.SemaphoreType.DMA,
            pltpu.SemaphoreType.DMA,
        ],
    )
    def sc_shift(x_hbm, out_hbm, buf0, buf1, rs0, rs1, ws0, ws1):
        c = lax.axis_index("c")
        s = lax.axis_index("s")
        wid = s * 2 + c
        row0 = wid * rows_per_w
        bufs = (buf0, buf1)
        rsem = (rs0, rs1)
        wsem = (ws0, ws1)
        read_h = {}
        write_h = {}

        def issue_read(ci):
            b = ci & 1
            r = row0 + ci * CHUNK
            h1 = pltpu.async_copy(
                x_hbm.at[pl.ds(r, CHUNK), pl.ds(0, body)],
                bufs[b].at[:, pl.ds(N_SHIFT, body)],
                rsem[b],
            )
            h2 = pltpu.async_copy(
                x_hbm.at[pl.ds(r, CHUNK), pl.ds(body, N_SHIFT)],
                bufs[b].at[:, pl.ds(0, N_SHIFT)],
                rsem[b],
            )
            read_h[ci] = (h1, h2)

        def issue_write(ci):
            b = ci & 1
            r = row0 + ci * CHUNK
            write_h[ci] = pltpu.async_copy(
                bufs[b], out_hbm.at[pl.ds(r, CHUNK), :], wsem[b]
            )

        issue_read(0)
        for ci in range(n_chunks):
            for h in read_h.pop(ci):
                h.wait()
            if ci + 1 < n_chunks:
                if ci >= 1:
                    write_h.pop(ci - 1).wait()
                issue_read(ci + 1)
            issue_write(ci)
        write_h.pop(n_chunks - 2).wait()
        write_h.pop(n_chunks - 1).wait()

    out = sc_shift(x.reshape(rows, T))
    return out.reshape(B, R, T)


# speedup vs baseline: 1.0077x; 1.0077x over previous
import functools

import jax
import jax.numpy as jnp
from jax import lax
from jax.experimental import pallas as pl
from jax.experimental.pallas import tpu as pltpu
from jax.experimental.pallas import tpu_sc as plsc

N_SHIFT = 128
RG = 8
NBUF = 2


def kernel(x):
    B, R, T = x.shape
    rows = B * R
    half = T // 2
    n_workers = 32
    rows_per_w = rows // n_workers
    n_groups = rows_per_w // RG
    n_chunks = 2 * n_groups

    mesh = plsc.VectorSubcoreMesh(core_axis_name="c", subcore_axis_name="s")

    @functools.partial(
        pl.kernel,
        mesh=mesh,
        out_type=jax.ShapeDtypeStruct((rows, T), jnp.float32),
        scratch_types=(
            [pltpu.VMEM((RG, half), jnp.float32) for _ in range(NBUF)]
            + [pltpu.SemaphoreType.DMA for _ in range(2 * NBUF)]
        ),
    )
    def sc_shift(x_hbm, out_hbm, *scratch):
        bufs = scratch[:NBUF]
        rsem = scratch[NBUF : 2 * NBUF]
        wsem = scratch[2 * NBUF :]
        c = lax.axis_index("c")
        s = lax.axis_index("s")
        wid = s * 2 + c
        row0 = wid * rows_per_w
        read_h = {}
        write_h = {}

        def issue_read(ci):
            b = ci % NBUF
            g, h = divmod(ci, 2)
            rs = pl.ds(row0 + g * RG, RG)
            if h == 0:
                h1 = pltpu.async_copy(
                    x_hbm.at[rs, pl.ds(0, half - N_SHIFT)],
                    bufs[b].at[:, pl.ds(N_SHIFT, half - N_SHIFT)],
                    rsem[b],
                )
                h2 = pltpu.async_copy(
                    x_hbm.at[rs, pl.ds(T - N_SHIFT, N_SHIFT)],
                    bufs[b].at[:, pl.ds(0, N_SHIFT)],
                    rsem[b],
                )
                read_h[ci] = (h1, h2)
            else:
                read_h[ci] = (
                    pltpu.async_copy(
                        x_hbm.at[rs, pl.ds(half - N_SHIFT, half)],
                        bufs[b],
                        rsem[b],
                    ),
                )

        def issue_write(ci):
            b = ci % NBUF
            g, h = divmod(ci, 2)
            rs = pl.ds(row0 + g * RG, RG)
            write_h[ci] = pltpu.async_copy(
                bufs[b], out_hbm.at[rs, pl.ds(h * half, half)], wsem[b]
            )

        for ci in range(min(NBUF - 1, n_chunks)):
            issue_read(ci)
        for ci in range(n_chunks):
            for h in read_h.pop(ci):
                h.wait()
            nxt = ci + NBUF - 1
            if nxt < n_chunks:
                if ci >= 1:
                    write_h.pop(ci - 1).wait()
                issue_read(nxt)
            issue_write(ci)
        for ci in sorted(write_h):
            write_h.pop(ci).wait()

    out = sc_shift(x.reshape(rows, T))
    return out.reshape(B, R, T)


# speedup vs baseline: 1.0886x; 1.0802x over previous
import functools

import jax
import jax.numpy as jnp
from jax import lax
from jax.experimental import pallas as pl
from jax.experimental.pallas import tpu as pltpu
from jax.experimental.pallas import tpu_sc as plsc

N_SHIFT = 128
RG = 4
NBUF = 2


def kernel(x):
    B, R, T = x.shape
    rows = B * R
    body = T - N_SHIFT
    n_workers = 32
    rows_per_w = rows // n_workers
    n_chunks = rows_per_w // RG

    mesh = plsc.VectorSubcoreMesh(core_axis_name="c", subcore_axis_name="s")

    @functools.partial(
        pl.kernel,
        mesh=mesh,
        out_type=jax.ShapeDtypeStruct((rows, T), jnp.float32),
        scratch_types=(
            [pltpu.VMEM_SHARED((16, NBUF, RG, T), jnp.float32)]
            + [pltpu.SemaphoreType.DMA for _ in range(2 * NBUF)]
        ),
    )
    def sc_shift(x_hbm, out_hbm, shared, *sems):
        rsem = sems[:NBUF]
        wsem = sems[NBUF:]
        c = lax.axis_index("c")
        s = lax.axis_index("s")
        wid = s * 2 + c
        row0 = wid * rows_per_w
        read_h = {}
        write_h = {}

        def issue_read(ci):
            b = ci % NBUF
            r = pl.ds(row0 + ci * RG, RG)
            h1 = pltpu.async_copy(
                x_hbm.at[r, pl.ds(0, body)],
                shared.at[s, b, :, pl.ds(N_SHIFT, body)],
                rsem[b],
            )
            h2 = pltpu.async_copy(
                x_hbm.at[r, pl.ds(body, N_SHIFT)],
                shared.at[s, b, :, pl.ds(0, N_SHIFT)],
                rsem[b],
            )
            read_h[ci] = (h1, h2)

        def issue_write(ci):
            b = ci % NBUF
            r = pl.ds(row0 + ci * RG, RG)
            write_h[ci] = pltpu.async_copy(
                shared.at[s, b], out_hbm.at[r, :], wsem[b]
            )

        issue_read(0)
        for ci in range(n_chunks):
            for h in read_h.pop(ci):
                h.wait()
            if ci + 1 < n_chunks:
                if ci >= 1:
                    write_h.pop(ci - 1).wait()
                issue_read(ci + 1)
            issue_write(ci)
        write_h.pop(n_chunks - 2).wait()
        write_h.pop(n_chunks - 1).wait()

    out = sc_shift(x.reshape(rows, T))
    return out.reshape(B, R, T)


# speedup vs baseline: 1.0932x; 1.0043x over previous
import functools

import jax
import jax.numpy as jnp
from jax import lax
from jax.experimental import pallas as pl
from jax.experimental.pallas import tpu as pltpu
from jax.experimental.pallas import tpu_sc as plsc

N_SHIFT = 128
RG = 4
NBUF = 3


def kernel(x):
    B, R, T = x.shape
    rows = B * R
    body = T - N_SHIFT
    n_workers = 32
    rows_per_w = rows // n_workers
    n_chunks = rows_per_w // RG

    mesh = plsc.VectorSubcoreMesh(core_axis_name="c", subcore_axis_name="s")

    @functools.partial(
        pl.kernel,
        mesh=mesh,
        out_type=jax.ShapeDtypeStruct((rows, T), jnp.float32),
        scratch_types=(
            [pltpu.VMEM_SHARED((16, NBUF, RG, T), jnp.float32)]
            + [pltpu.SemaphoreType.DMA for _ in range(2 * NBUF)]
        ),
    )
    def sc_shift(x_hbm, out_hbm, shared, *sems):
        rsem = sems[:NBUF]
        wsem = sems[NBUF:]
        c = lax.axis_index("c")
        s = lax.axis_index("s")
        wid = s * 2 + c
        row0 = wid * rows_per_w
        read_h = {}
        write_h = {}

        def issue_read(ci):
            b = ci % NBUF
            r = pl.ds(row0 + ci * RG, RG)
            h1 = pltpu.async_copy(
                x_hbm.at[r, pl.ds(0, body)],
                shared.at[s, b, :, pl.ds(N_SHIFT, body)],
                rsem[b],
            )
            h2 = pltpu.async_copy(
                x_hbm.at[r, pl.ds(body, N_SHIFT)],
                shared.at[s, b, :, pl.ds(0, N_SHIFT)],
                rsem[b],
            )
            read_h[ci] = (h1, h2)

        def issue_write(ci):
            b = ci % NBUF
            r = pl.ds(row0 + ci * RG, RG)
            write_h[ci] = pltpu.async_copy(
                shared.at[s, b], out_hbm.at[r, :], wsem[b]
            )

        for ci in range(min(NBUF - 1, n_chunks)):
            issue_read(ci)
        for ci in range(n_chunks):
            for h in read_h.pop(ci):
                h.wait()
            nxt = ci + NBUF - 1
            if nxt < n_chunks:
                if ci >= 1:
                    write_h.pop(ci - 1).wait()
                issue_read(nxt)
            issue_write(ci)
        for ci in sorted(write_h):
            write_h.pop(ci).wait()

    out = sc_shift(x.reshape(rows, T))
    return out.reshape(B, R, T)


# speedup vs baseline: 1.1163x; 1.0211x over previous
import functools

import jax
import jax.numpy as jnp
from jax import lax
from jax.experimental import pallas as pl
from jax.experimental.pallas import tpu as pltpu
from jax.experimental.pallas import tpu_sc as plsc

N_SHIFT = 128
RG = 8
NBUF = 2


def kernel(x):
    B, R, T = x.shape
    rows = B * R
    body = T - N_SHIFT
    n_workers = 32
    rows_per_w = rows // n_workers
    n_chunks = rows_per_w // RG

    mesh = plsc.VectorSubcoreMesh(core_axis_name="c", subcore_axis_name="s")

    @functools.partial(
        pl.kernel,
        mesh=mesh,
        out_type=jax.ShapeDtypeStruct((rows, T), jnp.float32),
        scratch_types=(
            [pltpu.VMEM_SHARED((16, NBUF, RG, T), jnp.float32)]
            + [pltpu.SemaphoreType.DMA for _ in range(2 * NBUF)]
        ),
    )
    def sc_shift(x_hbm, out_hbm, shared, *sems):
        rsem = sems[:NBUF]
        wsem = sems[NBUF:]
        c = lax.axis_index("c")
        s = lax.axis_index("s")
        wid = s * 2 + c
        row0 = wid * rows_per_w
        read_h = {}
        write_h = {}

        def issue_read(ci):
            b = ci % NBUF
            r = pl.ds(row0 + ci * RG, RG)
            h1 = pltpu.async_copy(
                x_hbm.at[r, pl.ds(0, body)],
                shared.at[s, b, :, pl.ds(N_SHIFT, body)],
                rsem[b],
            )
            h2 = pltpu.async_copy(
                x_hbm.at[r, pl.ds(body, N_SHIFT)],
                shared.at[s, b, :, pl.ds(0, N_SHIFT)],
                rsem[b],
            )
            read_h[ci] = (h1, h2)

        def issue_write(ci):
            b = ci % NBUF
            r = pl.ds(row0 + ci * RG, RG)
            write_h[ci] = pltpu.async_copy(
                shared.at[s, b], out_hbm.at[r, :], wsem[b]
            )

        for ci in range(min(NBUF - 1, n_chunks)):
            issue_read(ci)
        for ci in range(n_chunks):
            for h in read_h.pop(ci):
                h.wait()
            nxt = ci + NBUF - 1
            if nxt < n_chunks:
                if ci >= 1:
                    write_h.pop(ci - 1).wait()
                issue_read(nxt)
            issue_write(ci)
        for ci in sorted(write_h):
            write_h.pop(ci).wait()

    out = sc_shift(x.reshape(rows, T))
    return out.reshape(B, R, T)
